# baseline (speedup 1.0000x reference)
import functools

import jax
import jax.numpy as jnp
from jax.experimental import pallas as pl
from jax.experimental.pallas import tpu as pltpu

N = 100000
G = 1024
C = 256
NPAD = 102400
BIG = 1048576.0
PAD_POS = 1.0e9
PAD_GIDX = float(1 << 20)

_HIGH = jax.lax.Precision.HIGHEST


def _dot(a, b):
    return jax.lax.dot_general(a, b, (((1,), (0,)), ((), ())),
                               precision=_HIGH, preferred_element_type=jnp.float32)


def _dotT(a, b):
    return jax.lax.dot_general(a, b, (((0,), (0,)), ((), ())),
                               precision=_HIGH, preferred_element_type=jnp.float32)


def _celu(x):
    return jnp.where(x > 0.0, x, jnp.exp(jnp.minimum(x, 0.0)) - 1.0)


def _ln(x, g, b):
    m = jnp.mean(x, axis=-1, keepdims=True)
    v = jnp.mean((x - m) * (x - m), axis=-1, keepdims=True)
    return (x - m) * jax.lax.rsqrt(v + 1e-5) * g + b


def _min_kernel(pt_ref, out_ref):
    i = pl.program_id(0)

    @pl.when(i == 0)
    def _():
        out_ref[...] = jnp.full_like(out_ref, jnp.inf)

    blk = jnp.min(pt_ref[...], axis=0, keepdims=True)
    out_ref[...] = jnp.minimum(out_ref[...], blk)


def _glimpse_kernel(starts_ref, pt_hbm, minp_ref,
                    W1a, b1a, W1b, b1b, g1, be1,
                    W2a, b2a, W2b, b2b, g2, be2,
                    W3a, b3a, W3b, b3b, Wl, bl,
                    out_ref, ptw, sem):
    (W1a, b1a, W1b, b1b, g1, be1, W2a, b2a, W2b, b2b, g2, be2,
     W3a, b3a, W3b, b3b, Wl, bl) = (r[...] for r in (
        W1a, b1a, W1b, b1b, g1, be1, W2a, b2a, W2b, b2b, g2, be2,
        W3a, b3a, W3b, b3b, Wl, bl))
    g = pl.program_id(0)
    s = starts_ref[g]
    r0 = jnp.minimum(s, NPAD - C)
    cp = pltpu.make_async_copy(pt_hbm.at[pl.ds(r0, C), :], ptw, sem)
    cp.start()
    cp.wait()

    pt = ptw[...]
    pos = pt[:, 0:3]
    rgb = pt[:, 3:4]
    gf = pt[:, 4:5]
    gval = g.astype(jnp.float32)
    m = (gf == gval).astype(jnp.float32)

    mp = minp_ref[0:1, 0:3] - 0.5

    eye = (jax.lax.broadcasted_iota(jnp.int32, (C, C), 0) ==
           jax.lax.broadcasted_iota(jnp.int32, (C, C), 1)).astype(jnp.float32)

    def enc_key(v):
        return v[:, 0:1] * 4096.0 + v[:, 1:2] * 64.0 + v[:, 2:3]

    def eq_outer(kcol):
        krow = _dotT(kcol, eye)
        return (kcol == krow).astype(jnp.float32)

    v1 = jnp.floor((pos - mp) * 4.0)
    k1 = enc_key(v1) + (1.0 - m) * BIG
    E1 = eq_outer(k1)
    n1 = jnp.sum(E1, axis=1, keepdims=True)
    P1 = _dot(E1, pos) / n1
    h1 = _celu(_dot(jnp.concatenate([rgb, pos - P1], axis=1), W1a) + b1a)
    A1 = _dot(E1, h1) / n1
    F1 = _ln(_celu(_dot(A1, W1b) + b1b), g1, be1)

    v2 = jnp.floor((P1 - mp) * 2.0)
    k2 = enc_key(v2) + (1.0 - m) * BIG
    E2 = eq_outer(k2)
    w1 = 1.0 / n1
    den2 = _dot(E2, w1)
    P2 = _dot(E2, w1 * P1) / den2
    h2 = _celu(_dot(jnp.concatenate([F1, P1 - P2], axis=1), W2a) + b2a)
    A2 = _dot(E2, w1 * h2) / den2
    F2 = _ln(_celu(_dot(A2, W2b) + b2b), g2, be2)

    h3 = _celu(_dot(jnp.concatenate([F2, P2], axis=1), W3a) + b3a)
    n2 = jnp.sum(E2, axis=1, keepdims=True)
    w2 = m / n2
    s3 = _dotT(w2, h3)
    c3 = _dotT(w2, m)
    A3 = s3 / jnp.maximum(c3, 1.0)
    o = _dot(_celu(_dot(A3, W3b) + b3b), Wl) + bl
    out_ref[...] = o[None]


@jax.jit
def kernel(rgb, pos, glimpse_member__glimpse_index, glimpse__center, glimpse__batch,
           W1a, b1a, W1b, b1b, g1, be1,
           W2a, b2a, W2b, b2b, g2, be2,
           W3a, b3a, W3b, b3b, Wl, bl):
    gidx = glimpse_member__glimpse_index
    f32 = jnp.float32

    pt = jnp.concatenate([pos, rgb, gidx.astype(f32)[:, None],
                          jnp.zeros((N, 3), f32)], axis=1)
    padrow = jnp.array([[PAD_POS, PAD_POS, PAD_POS, 0.0, PAD_GIDX, 0.0, 0.0, 0.0]], f32)
    pt = jnp.concatenate([pt, jnp.broadcast_to(padrow, (NPAD - N, 8))], axis=0)

    starts = jnp.searchsorted(gidx, jnp.arange(G, dtype=jnp.int32)).astype(jnp.int32)

    minp = pl.pallas_call(
        _min_kernel,
        grid=(NPAD // 2048,),
        in_specs=[pl.BlockSpec((2048, 8), lambda i: (i, 0))],
        out_specs=pl.BlockSpec((1, 8), lambda i: (0, 0)),
        out_shape=jax.ShapeDtypeStruct((1, 8), f32),
    )(pt)

    b1a_, b1b_, g1_, be1_ = (x[None, :] for x in (b1a, b1b, g1, be1))
    b2a_, b2b_, g2_, be2_ = (x[None, :] for x in (b2a, b2b, g2, be2))
    b3a_, b3b_, bl_ = (x[None, :] for x in (b3a, b3b, bl))

    vmem = pl.BlockSpec(memory_space=pltpu.MemorySpace.VMEM)
    grid_spec = pltpu.PrefetchScalarGridSpec(
        num_scalar_prefetch=1,
        grid=(G,),
        in_specs=[pl.BlockSpec(memory_space=pltpu.MemorySpace.HBM)] +
                 [vmem] * 19,
        out_specs=pl.BlockSpec((1, 1, 256), lambda g, starts: (g, 0, 0)),
        scratch_shapes=[pltpu.VMEM((C, 8), f32), pltpu.SemaphoreType.DMA],
    )

    out = pl.pallas_call(
        _glimpse_kernel,
        grid_spec=grid_spec,
        out_shape=jax.ShapeDtypeStruct((G, 1, 256), f32),
    )(starts, pt, minp,
      W1a, b1a_, W1b, b1b_, g1_, be1_,
      W2a, b2a_, W2b, b2b_, g2_, be2_,
      W3a, b3a_, W3b, b3b_, Wl, bl_)
    return out.reshape(G, 256)


# speedup vs baseline: 3.7956x; 3.7956x over previous
import functools

import jax
import jax.numpy as jnp
from jax import lax
from jax.experimental import pallas as pl
from jax.experimental.pallas import tpu as pltpu
from jax.experimental.pallas import tpu_sc as plsc

N = 100000
G = 1024
C = 128
GPB = 4
NPAD = 102400
PAD_POS = 1.0e9
PAD_GIDX = float(1 << 20)
WOFF = float(1 << 18)

def _bdot(a, b):
    return jax.lax.dot_general(a, b, (((1,), (0,)), ((), ())),
                               preferred_element_type=jnp.float32)


def _split2(x):
    hi = x.astype(jnp.bfloat16)
    lo = (x - hi.astype(jnp.float32)).astype(jnp.bfloat16)
    return hi, lo


def _split3(x):
    hi = x.astype(jnp.bfloat16)
    r = x - hi.astype(jnp.float32)
    mid = r.astype(jnp.bfloat16)
    lo = (r - mid.astype(jnp.float32)).astype(jnp.bfloat16)
    return hi, mid, lo


def _dotE_exact(Ebf, x):
    h, m, l = _split3(x)
    w = x.shape[1]
    y = _bdot(Ebf, jnp.concatenate([h, m, l], axis=1))
    return y[:, 0:w] + y[:, w:2 * w] + y[:, 2 * w:3 * w]


def _dotE(Ebf, x):
    h, l = _split2(x)
    w = x.shape[1]
    y = _bdot(Ebf, jnp.concatenate([h, l], axis=1))
    return y[:, 0:w] + y[:, w:2 * w]


def _dotW(a, wstk):
    h, l = _split2(a)
    return _bdot(jnp.concatenate([h, h, l], axis=1), wstk)


def _dotT_packed(a, b):
    ah, al = _split2(a)
    bh, bl = _split2(b)
    return jax.lax.dot_general(
        jnp.concatenate([ah, ah, al], axis=0),
        jnp.concatenate([bh, bl, bh], axis=0),
        (((0,), (0,)), ((), ())), preferred_element_type=jnp.float32)


def _celu(x):
    return jnp.where(x > 0.0, x, jnp.exp(jnp.minimum(x, 0.0)) - 1.0)


def _ln(x, g, b):
    m = jnp.mean(x, axis=-1, keepdims=True)
    v = jnp.mean((x - m) * (x - m), axis=-1, keepdims=True)
    return (x - m) * jax.lax.rsqrt(v + 1e-5) * g + b


NP2 = 102400
_CHUNK = NP2 // 8


@functools.partial(
    pl.kernel,
    mesh=plsc.VectorSubcoreMesh(core_axis_name="c", subcore_axis_name="s"),
    out_type=jax.ShapeDtypeStruct((24, 16), jnp.float32),
    scratch_types=[pltpu.VMEM((_CHUNK,), jnp.float32),
                   pltpu.VMEM((16,), jnp.float32)],
)
def _sc_min(post_hbm, out_hbm, chunk_v, part_v):
    wid = lax.axis_index("s") * 2 + lax.axis_index("c")

    @pl.when(wid < 24)
    def _():
        pltpu.sync_copy(post_hbm.at[pl.ds(wid * _CHUNK, _CHUNK)], chunk_v)

        def body(j, acc):
            return jnp.minimum(acc, chunk_v[pl.ds(j * 16, 16)])

        acc = lax.fori_loop(0, _CHUNK // 16,
                            body, jnp.full((16,), jnp.inf, jnp.float32))
        part_v[pl.ds(0, 16)] = acc
        pltpu.sync_copy(part_v, out_hbm.at[wid])


def _glimpse_compute(pt, gbase, mp,
                     W1a, b1a, W1b, b1b, g1, be1,
                     W2a, b2a, W2b, b2b, g2, be2,
                     W3a, b3a, W3b, b3b, Wl, bl):
    pos = jnp.minimum(pt[:, 0:3], 100.0)
    rgb = pt[:, 3:4]
    gf = pt[:, 4:5]
    relf = gf - gbase
    m = ((relf >= 0.0) & (relf <= GPB - 1.0)).astype(jnp.float32)

    def enc_key(v):
        return v[:, 0:1] * 4096.0 + v[:, 1:2] * 64.0 + v[:, 2:3]

    ones = jnp.ones((GPB * C, 1), jnp.float32)

    def eq_outer(kcol):
        krow = jnp.transpose(kcol, (1, 0))
        return (kcol == krow).astype(jnp.bfloat16)

    v1 = jnp.floor((pos - mp) * 4.0)
    k1 = enc_key(v1) + relf * WOFF
    E1bf = eq_outer(k1)
    S1 = _dotE_exact(E1bf, jnp.concatenate([pos, ones], axis=1))
    n1 = S1[:, 3:4]
    P1 = S1[:, 0:3] / n1
    h1 = _celu(_dotW(jnp.concatenate([rgb, pos - P1], axis=1), W1a) + b1a)
    A1 = _dotE(E1bf, h1) / n1
    F1 = _ln(_celu(_dotW(A1, W1b) + b1b), g1, be1)

    v2 = jnp.floor((P1 - mp) * 2.0)
    k2 = enc_key(v2) + relf * WOFF
    E2bf = eq_outer(k2)
    w1 = 1.0 / n1
    S2 = _dotE(E2bf, jnp.concatenate([w1, w1 * P1, ones], axis=1))
    den2 = S2[:, 0:1]
    P2 = S2[:, 1:4] / den2
    n2 = S2[:, 4:5]
    h2 = _celu(_dotW(jnp.concatenate([F1, P1 - P2], axis=1), W2a) + b2a)
    A2 = _dotE(E2bf, w1 * h2) / den2
    F2 = _ln(_celu(_dotW(A2, W2b) + b2b), g2, be2)

    h3 = _celu(_dotW(jnp.concatenate([F2, P2], axis=1), W3a) + b3a)
    w2 = m / n2
    wsel = (relf == jax.lax.broadcasted_iota(
        jnp.int32, (1, GPB), 1).astype(jnp.float32))
    w2c = w2 * wsel.astype(jnp.float32)
    sc3 = _dotT_packed(w2c, jnp.concatenate([h3, m], axis=1))
    s3 = sc3[:, 0:128]
    c3 = sc3[:, 128:129]
    A3 = s3 / jnp.maximum(c3, 1.0)
    return _dotW(_celu(_dotW(A3, W3b) + b3b), Wl) + bl


def _glimpse_kernel(starts_ref, pt_hbm, minp_ref,
                    W1a, b1a, W1b, b1b, g1, be1,
                    W2a, b2a, W2b, b2b, g2, be2,
                    W3a, b3a, W3b, b3b, Wl, bl,
                    out_ref, ptw, sems):
    weights = tuple(r[...] for r in (
        W1a, b1a, W1b, b1b, g1, be1, W2a, b2a, W2b, b2b, g2, be2,
        W3a, b3a, W3b, b3b, Wl, bl))
    i = pl.program_id(0)
    nsteps = pl.num_programs(0)
    slot = jax.lax.rem(i, 2)

    def start_fetch(step, buf):
        s = starts_ref[step * GPB]
        r0 = jnp.minimum(s, NPAD - GPB * C)
        pltpu.make_async_copy(pt_hbm.at[pl.ds(r0, GPB * C), :],
                              ptw.at[buf], sems.at[buf]).start()

    @pl.when(i == 0)
    def _():
        start_fetch(0, 0)

    @pl.when(i + 1 < nsteps)
    def _():
        start_fetch(i + 1, 1 - slot)

    pltpu.make_async_copy(pt_hbm.at[pl.ds(0, GPB * C), :],
                          ptw.at[slot], sems.at[slot]).wait()

    sm = minp_ref[...]
    mp = jnp.concatenate(
        [jnp.full((1, 1), jnp.min(sm[8 * r:8 * (r + 1), :]), jnp.float32)
         for r in range(3)], axis=1) - 0.5
    gbase = (i * GPB).astype(jnp.float32)
    o = _glimpse_compute(ptw[slot], gbase, mp, *weights)
    out_ref[0] = o


@jax.jit
def kernel(rgb, pos, glimpse_member__glimpse_index, glimpse__center, glimpse__batch,
           W1a, b1a, W1b, b1b, g1, be1,
           W2a, b2a, W2b, b2b, g2, be2,
           W3a, b3a, W3b, b3b, Wl, bl):
    gidx = glimpse_member__glimpse_index
    f32 = jnp.float32

    pt = jnp.concatenate([pos, rgb, gidx.astype(f32)[:, None],
                          jnp.zeros((N, 3), f32)], axis=1)
    padrow = jnp.array([[PAD_POS, PAD_POS, PAD_POS, 0.0, PAD_GIDX, 0.0, 0.0, 0.0]], f32)
    pt = jnp.concatenate([pt, jnp.broadcast_to(padrow, (NPAD - N, 8))], axis=0)

    starts = jnp.searchsorted(gidx, jnp.arange(G, dtype=jnp.int32)).astype(jnp.int32)

    post = jnp.concatenate(
        [pos.T, jnp.full((3, NP2 - N), PAD_POS, f32)], axis=1).reshape(3 * NP2)
    minp = _sc_min(post)

    b1a_, b1b_, g1_, be1_ = (x[None, :] for x in (b1a, b1b, g1, be1))
    b2a_, b2b_, g2_, be2_ = (x[None, :] for x in (b2a, b2b, g2, be2))
    b3a_, b3b_, bl_ = (x[None, :] for x in (b3a, b3b, bl))

    def wstack(w):
        hi = w.astype(jnp.bfloat16)
        lo = (w - hi.astype(f32)).astype(jnp.bfloat16)
        return jnp.concatenate([hi, lo, hi], axis=0)

    W1a, W1b, W2a, W2b, W3a, W3b, Wl = (
        wstack(w) for w in (W1a, W1b, W2a, W2b, W3a, W3b, Wl))

    vmem = pl.BlockSpec(memory_space=pltpu.MemorySpace.VMEM)
    grid_spec = pltpu.PrefetchScalarGridSpec(
        num_scalar_prefetch=1,
        grid=(G // GPB,),
        in_specs=[pl.BlockSpec(memory_space=pltpu.MemorySpace.HBM)] +
                 [vmem] * 19,
        out_specs=pl.BlockSpec((1, GPB, 256), lambda i, starts: (i, 0, 0)),
        scratch_shapes=[pltpu.VMEM((2, GPB * C, 8), f32),
                        pltpu.SemaphoreType.DMA((2,))],
    )

    out = pl.pallas_call(
        _glimpse_kernel,
        grid_spec=grid_spec,
        out_shape=jax.ShapeDtypeStruct((G // GPB, GPB, 256), f32),
    )(starts, pt, minp,
      W1a, b1a_, W1b, b1b_, g1_, be1_,
      W2a, b2a_, W2b, b2b_, g2_, be2_,
      W3a, b3a_, W3b, b3b_, Wl, bl_)
    return out.reshape(G, 256)
